# baseline (speedup 1.0000x reference)
import functools

import jax
import jax.numpy as jnp
from jax import lax
from jax.experimental import pallas as pl
from jax.experimental.pallas import tpu as pltpu
from jax.experimental.pallas import tpu_sc as plsc

H = 128
N = 10000
E = 320000
R2E = 2 * E
BN_EPS = 1e-5

NC, NS = 2, 16
NW = NC * NS
CH = 128
NCHUNK = R2E // CH
NK_CORE = -(-NCHUNK // NS)
NK_ALL = -(-NCHUNK // NW)
NT = 10240
NPT = NT // NS

BK = 2560
GH = -(-(R2E // 2) // BK)
GSTEPS = R2E // BK


def _sc_scatter(edge_rep, inter, swap, znh, oneh):
    mesh = plsc.VectorSubcoreMesh(core_axis_name="c", subcore_axis_name="s")

    @functools.partial(
        pl.kernel,
        out_type=(
            jax.ShapeDtypeStruct((NT, H), jnp.float32),
            jax.ShapeDtypeStruct((NT, H), jnp.float32),
            jax.ShapeDtypeStruct((NT, H), jnp.float32),
            jax.ShapeDtypeStruct((NT, H), jnp.float32),
        ),
        mesh=mesh,
        scratch_types=[
            pltpu.VMEM_SHARED((NT, H), jnp.float32),
            pltpu.VMEM((CH, H), jnp.float32),
            pltpu.VMEM((CH, H), jnp.float32),
            pltpu.VMEM((CH,), jnp.int32),
            pltpu.VMEM((CH,), jnp.int32),
            pltpu.SemaphoreType.DMA,
            pltpu.SemaphoreType.DMA,
            pltpu.SemaphoreType.DMA,
            pltpu.SemaphoreType.DMA,
        ],
    )
    def k(er_hbm, inter_hbm, swap_hbm, znh_hbm, oneh_hbm,
          ns1_hbm, cross_hbm, dega_hbm, degb_hbm,
          tab_sh, vrows0, vrows1, vidx0, vidx1,
          sr0, sr1, si0, si1):
        c = lax.axis_index("c")
        s = lax.axis_index("s")
        wid = s * NC + c
        base = s * NPT
        vrows = vrows0

        def zero_table():
            pltpu.sync_copy(znh_hbm, vrows)
            for j in range(NPT // CH):
                pltpu.sync_copy(vrows, tab_sh.at[pl.ds(base + j * CH, CH)])

        zero_table()
        plsc.subcore_barrier()

        def make_row_phase(idx_hbm):
            bufs = ((vrows0, vidx0, sr0, si0), (vrows1, vidx1, sr1, si1))

            def start(mm, b):
                vr, vi, sr, si = bufs[b]

                @pl.when(mm < NCHUNK)
                def _():
                    r0 = mm * CH
                    pltpu.async_copy(er_hbm.at[pl.ds(r0, CH)], vr, sr)
                    pltpu.async_copy(idx_hbm.at[pl.ds(r0, CH)], vi, si)

            def finish(mm, b):
                vr, vi, sr, si = bufs[b]

                @pl.when(mm < NCHUNK)
                def _():
                    pltpu.make_async_copy(
                        er_hbm.at[pl.ds(0, CH)], vr, sr).wait()
                    pltpu.make_async_copy(
                        idx_hbm.at[pl.ds(0, CH)], vi, si).wait()
                    pltpu.sync_copy(vr, tab_sh.at[vi], add=True)

            def run():
                start(s, 0)

                def body(t2, carry):
                    m0 = s + (2 * t2) * NS
                    start(m0 + NS, 1)
                    finish(m0, 0)
                    start(m0 + 2 * NS, 0)
                    finish(m0 + NS, 1)
                    return carry

                lax.fori_loop(0, -(-NK_CORE // 2), body, 0)

            return run

        @pl.when(c == 0)
        def _():
            make_row_phase(inter_hbm)()

        @pl.when(c == 1)
        def _():
            make_row_phase(swap_hbm)()

        plsc.subcore_barrier()

        def write_table(dst_hbm):
            for j in range(NPT // CH):
                sl = pl.ds(base + j * CH, CH)
                pltpu.sync_copy(tab_sh.at[sl], vrows)
                pltpu.sync_copy(vrows, dst_hbm.at[sl])

        @pl.when(c == 0)
        def _():
            write_table(ns1_hbm)

        @pl.when(c == 1)
        def _():
            write_table(cross_hbm)

        plsc.subcore_barrier()
        zero_table()
        pltpu.sync_copy(oneh_hbm, vrows0)
        plsc.subcore_barrier()

        def make_deg_body(idx_hbm):
            def body(kk, carry):
                m = wid + kk * NW

                @pl.when(m < NCHUNK)
                def _():
                    r0 = m * CH
                    pltpu.sync_copy(idx_hbm.at[pl.ds(r0, CH)], vidx0)
                    pltpu.sync_copy(vrows0, tab_sh.at[vidx0], add=True)

                return carry

            return body

        @pl.when(c == 0)
        def _():
            lax.fori_loop(0, NK_ALL, make_deg_body(inter_hbm), 0)

        @pl.when(c == 1)
        def _():
            lax.fori_loop(0, NK_ALL, make_deg_body(swap_hbm), 0)

        plsc.subcore_barrier()

        @pl.when(c == 0)
        def _():
            write_table(dega_hbm)

        @pl.when(c == 1)
        def _():
            write_table(degb_hbm)

    return k(edge_rep, inter, swap, znh, oneh)


NRG = R2E // 2
NCHUNK_G = NRG // CH
NKG = -(-NCHUNK_G // NW)


def _sc_gather(P, Q, inter, swap):
    mesh = plsc.VectorSubcoreMesh(core_axis_name="c", subcore_axis_name="s")

    @functools.partial(
        pl.kernel,
        out_type=jax.ShapeDtypeStruct((NRG, H), jnp.float32),
        mesh=mesh,
        scratch_types=[
            pltpu.VMEM((CH,), jnp.int32),
            pltpu.VMEM((CH,), jnp.int32),
            pltpu.VMEM((CH,), jnp.int32),
            pltpu.VMEM((CH,), jnp.int32),
            pltpu.VMEM((CH, H), jnp.float32),
            pltpu.VMEM((CH, H), jnp.float32),
            pltpu.VMEM((CH, H), jnp.float32),
            pltpu.VMEM((CH, H), jnp.float32),
            pltpu.SemaphoreType.DMA,
            pltpu.SemaphoreType.DMA,
            pltpu.SemaphoreType.DMA,
            pltpu.SemaphoreType.DMA,
        ],
    )
    def k(p_hbm, q_hbm, inter_hbm, swap_hbm, r_hbm,
          vidxa0, vidxb0, vidxa1, vidxb1, bufp0, bufq0, bufp1, bufq1,
          sema0, semb0, sema1, semb1):
        c = lax.axis_index("c")
        s = lax.axis_index("s")
        wid = s * NC + c
        bufs = ((vidxa0, vidxb0, bufp0, bufq0, sema0, semb0),
                (vidxa1, vidxb1, bufp1, bufq1, sema1, semb1))

        def start(mm, b):
            ia, ib, bp, bq, sa, sb = bufs[b]

            @pl.when(mm < NCHUNK_G)
            def _():
                r0 = mm * CH
                pltpu.sync_copy(inter_hbm.at[pl.ds(r0, CH)], ia)
                pltpu.sync_copy(swap_hbm.at[pl.ds(r0, CH)], ib)
                pltpu.async_copy(p_hbm.at[ia], bp, sa)
                pltpu.async_copy(q_hbm.at[ib], bq, sb)

        def finish(mm, b):
            ia, ib, bp, bq, sa, sb = bufs[b]

            @pl.when(mm < NCHUNK_G)
            def _():
                pltpu.make_async_copy(p_hbm.at[ia], bp, sa).wait()
                pltpu.make_async_copy(q_hbm.at[ib], bq, sb).wait()

                def addrow(r, cr):
                    for j in range(H // 16):
                        slj = pl.ds(j * 16, 16)
                        bp[r, slj] = bp[r, slj] + bq[r, slj]
                    return cr

                lax.fori_loop(0, CH, addrow, 0)
                pltpu.sync_copy(bp, r_hbm.at[pl.ds(mm * CH, CH)])

        start(wid, 0)

        def body(t2, carry):
            m0 = wid + (2 * t2) * NW
            start(m0 + NW, 1)
            finish(m0, 0)
            start(m0 + 2 * NW, 0)
            finish(m0 + NW, 1)
            return carry

        lax.fori_loop(0, -(-NKG // 2), body, 0)

    return k(P, Q, inter, swap)


def _node_body(ns1_ref, cross_ref, dega_ref, degb_ref, e0_ref,
               wc1_ref, wc12_ref, wa12_ref, p_ref, q_ref):
    ns1 = ns1_ref[...]
    dcol = jnp.dot(dega_ref[...] + degb_ref[...], e0_ref[...],
                   preferred_element_type=jnp.float32)
    p_ref[...] = (dcol *
                  jnp.dot(ns1, wc12_ref[...], preferred_element_type=jnp.float32)
                  + jnp.dot(cross_ref[...], wc1_ref[...],
                            preferred_element_type=jnp.float32))
    q_ref[...] = jnp.dot(ns1, wa12_ref[...], preferred_element_type=jnp.float32)


BNT = 2048


def _tc_node(ns1, cross, dega, degb, Wc1, Wc12, Wa12):
    e0 = jnp.zeros((H, 1), jnp.float32).at[0, 0].set(1.0)
    return pl.pallas_call(
        _node_body,
        grid=(NT // BNT,),
        in_specs=[
            pl.BlockSpec((BNT, H), lambda i: (i, 0)),
            pl.BlockSpec((BNT, H), lambda i: (i, 0)),
            pl.BlockSpec((BNT, H), lambda i: (i, 0)),
            pl.BlockSpec((BNT, H), lambda i: (i, 0)),
            pl.BlockSpec((H, 1), lambda i: (0, 0)),
            pl.BlockSpec((H, H), lambda i: (0, 0)),
            pl.BlockSpec((H, H), lambda i: (0, 0)),
            pl.BlockSpec((H, H), lambda i: (0, 0)),
        ],
        out_specs=(
            pl.BlockSpec((BNT, H), lambda i: (i, 0)),
            pl.BlockSpec((BNT, H), lambda i: (i, 0)),
        ),
        out_shape=(
            jax.ShapeDtypeStruct((NT, H), jnp.float32),
            jax.ShapeDtypeStruct((NT, H), jnp.float32),
        ),
        compiler_params=pltpu.CompilerParams(
            dimension_semantics=("parallel",)),
    )(ns1, cross, dega, degb, e0, Wc1, Wc12, Wa12)


def _p2_body(er_ref, r_ref, wa1_ref, b_ref, scale_ref, sy_ref, gy_ref):
    i = pl.program_id(0)
    er = er_ref[...]
    y = (jnp.dot(er, wa1_ref[...], preferred_element_type=jnp.float32)
         + r_ref[...] + b_ref[...] + scale_ref[0, 0] * er)

    @pl.when(i == 0)
    def _():
        sy_ref[...] = jnp.zeros_like(sy_ref)
        gy_ref[...] = jnp.zeros_like(gy_ref)

    sy_ref[...] += jnp.sum(y, axis=0, keepdims=True)
    gy_ref[...] += lax.dot_general(y, y, (((0,), (0,)), ((), ())),
                                   preferred_element_type=jnp.float32)


def _tc_p2(edge_rep, R_half, Wa1, b_lin, scale, half):
    off = half * GH
    return pl.pallas_call(
        _p2_body,
        grid=(GH,),
        in_specs=[
            pl.BlockSpec((BK, H), lambda i: (off + i, 0)),
            pl.BlockSpec((BK, H), lambda i: (i, 0)),
            pl.BlockSpec((H, H), lambda i: (0, 0)),
            pl.BlockSpec((1, H), lambda i: (0, 0)),
            pl.BlockSpec(memory_space=pltpu.SMEM),
        ],
        out_specs=(
            pl.BlockSpec((1, H), lambda i: (0, 0)),
            pl.BlockSpec((H, H), lambda i: (0, 0)),
        ),
        out_shape=(
            jax.ShapeDtypeStruct((1, H), jnp.float32),
            jax.ShapeDtypeStruct((H, H), jnp.float32),
        ),
        compiler_params=pltpu.CompilerParams(
            dimension_semantics=("arbitrary",)),
    )(edge_rep, R_half, Wa1, b_lin, scale)


def _stats_body(sya_ref, syb_ref, gya_ref, gyb_ref, w_ref,
                gamma_ref, beta_ref, s_ref, t_ref):
    w = w_ref[...]
    inv_n = 1.0 / float(R2E)
    sum_v = sya_ref[...] + syb_ref[...]
    gram = gya_ref[...] + gyb_ref[...]
    mu = jnp.dot(sum_v, w, preferred_element_type=jnp.float32) * inv_n
    gw = jnp.dot(gram, w, preferred_element_type=jnp.float32)
    e2 = jnp.sum(w * gw, axis=0, keepdims=True) * inv_n
    var = e2 - mu * mu
    s = gamma_ref[...] * lax.rsqrt(var + BN_EPS)
    s_ref[...] = s
    t_ref[...] = beta_ref[...] - mu * s


def _tc_stats(sya, syb, gya, gyb, W, gamma, beta):
    d = W.shape[1]
    return pl.pallas_call(
        _stats_body,
        out_shape=(
            jax.ShapeDtypeStruct((1, d), jnp.float32),
            jax.ShapeDtypeStruct((1, d), jnp.float32),
        ),
    )(sya, syb, gya, gyb, W, gamma, beta)


def _p3_body(er_ref, ra_ref, rb_ref, wa1_ref, b_ref, scale_ref,
             w1t_ref, s1_ref, t1_ref, w2t_ref,
             h3_ref, sh_ref, qh_ref):
    g = pl.program_id(0)
    i = pl.program_id(1)
    er = er_ref[...]
    ya = (jnp.dot(er, wa1_ref[...], preferred_element_type=jnp.float32)
          + b_ref[...] + scale_ref[0, 0] * er)
    r = jnp.where(g == 0, ra_ref[...], rb_ref[...])
    y = ya + r
    h = jnp.dot(y, w1t_ref[...], preferred_element_type=jnp.float32)
    z = jnp.maximum(h * s1_ref[...] + t1_ref[...], 0.0)
    h3 = jnp.dot(z, w2t_ref[...], preferred_element_type=jnp.float32)
    h3_ref[...] = h3

    @pl.when(jnp.logical_and(g == 0, i == 0))
    def _():
        sh_ref[...] = jnp.zeros_like(sh_ref)
        qh_ref[...] = jnp.zeros_like(qh_ref)

    sh_ref[...] += jnp.sum(h3, axis=0, keepdims=True)
    qh_ref[...] += jnp.sum(h3 * h3, axis=0, keepdims=True)


def _tc_p3(edge_rep, Ra, Rb, Wa1, b_lin, scale, W1t, s1, t1, W2t):
    return pl.pallas_call(
        _p3_body,
        grid=(2, GH),
        in_specs=[
            pl.BlockSpec((BK, H), lambda g, i: (g * GH + i, 0)),
            pl.BlockSpec((BK, H), lambda g, i: (jnp.where(g == 0, i, 0), 0)),
            pl.BlockSpec((BK, H), lambda g, i: (jnp.where(g == 0, 0, i), 0)),
            pl.BlockSpec((H, H), lambda g, i: (0, 0)),
            pl.BlockSpec((1, H), lambda g, i: (0, 0)),
            pl.BlockSpec(memory_space=pltpu.SMEM),
            pl.BlockSpec((H, 2 * H), lambda g, i: (0, 0)),
            pl.BlockSpec((1, 2 * H), lambda g, i: (0, 0)),
            pl.BlockSpec((1, 2 * H), lambda g, i: (0, 0)),
            pl.BlockSpec((2 * H, H), lambda g, i: (0, 0)),
        ],
        out_specs=(
            pl.BlockSpec((BK, H), lambda g, i: (g * GH + i, 0)),
            pl.BlockSpec((1, H), lambda g, i: (0, 0)),
            pl.BlockSpec((1, H), lambda g, i: (0, 0)),
        ),
        out_shape=(
            jax.ShapeDtypeStruct((R2E, H), jnp.float32),
            jax.ShapeDtypeStruct((1, H), jnp.float32),
            jax.ShapeDtypeStruct((1, H), jnp.float32),
        ),
        compiler_params=pltpu.CompilerParams(
            dimension_semantics=("arbitrary", "arbitrary")),
    )(edge_rep, Ra, Rb, Wa1, b_lin, scale, W1t, s1, t1, W2t)


def _stats2_body(sh_ref, qh_ref, gamma_ref, beta_ref, s_ref, t_ref):
    inv_n = 1.0 / float(R2E)
    mu = sh_ref[...] * inv_n
    var = qh_ref[...] * inv_n - mu * mu
    s = gamma_ref[...] * lax.rsqrt(var + BN_EPS)
    s_ref[...] = s
    t_ref[...] = beta_ref[...] - mu * s


def _tc_stats2(sh, qh, gamma, beta):
    return pl.pallas_call(
        _stats2_body,
        out_shape=(
            jax.ShapeDtypeStruct((1, H), jnp.float32),
            jax.ShapeDtypeStruct((1, H), jnp.float32),
        ),
    )(sh, qh, gamma, beta)


def _p4_body(h3_ref, s2_ref, t2_ref, out_ref):
    out_ref[...] = jnp.maximum(h3_ref[...] * s2_ref[...] + t2_ref[...], 0.0)


def _tc_p4(h3, s2, t2):
    return pl.pallas_call(
        _p4_body,
        grid=(GSTEPS,),
        in_specs=[
            pl.BlockSpec((BK, H), lambda i: (i, 0)),
            pl.BlockSpec((1, H), lambda i: (0, 0)),
            pl.BlockSpec((1, H), lambda i: (0, 0)),
        ],
        out_specs=pl.BlockSpec((BK, H), lambda i: (i, 0)),
        out_shape=jax.ShapeDtypeStruct((R2E, H), jnp.float32),
        compiler_params=pltpu.CompilerParams(
            dimension_semantics=("parallel",)),
    )(h3, s2, t2)


def kernel(edge_rep, edge_index, W_lin, b_lin, W1, gamma1, beta1,
           W2, gamma2, beta2, eps):
    src = edge_index[0]
    dst = edge_index[1]
    inter = jnp.stack([src, dst], axis=1).reshape(-1)
    swap = jnp.stack([dst, src], axis=1).reshape(-1)

    Wt = W_lin.T
    Wa1, Wa2 = Wt[0:H], Wt[H:2 * H]
    Wb1, Wb2 = Wt[2 * H:3 * H], Wt[3 * H:4 * H]
    Wc1 = Wa1 + Wb1
    Wc12 = Wc1 + Wa2 + Wb2
    Wa12 = Wa1 + Wa2

    znh = jnp.zeros((CH, H), jnp.float32)
    oneh = jnp.ones((CH, H), jnp.float32)

    ns1, cross, dega, degb = _sc_scatter(edge_rep, inter, swap, znh, oneh)
    P, Q = _tc_node(ns1, cross, dega, degb, Wc1, Wc12, Wa12)

    Ra = _sc_gather(P, Q, inter[:NRG], swap[:NRG])
    Rb = _sc_gather(P, Q, inter[NRG:], swap[NRG:])

    scale = (1.0 + eps).reshape(1, 1)
    b_row = b_lin.reshape(1, H)
    sya, gya = _tc_p2(edge_rep, Ra, Wa1, b_row, scale, 0)
    syb, gyb = _tc_p2(edge_rep, Rb, Wa1, b_row, scale, 1)

    W1t = W1.T
    W2t = W2.T
    s1, t1 = _tc_stats(sya, syb, gya, gyb, W1t, gamma1.reshape(1, -1),
                       beta1.reshape(1, -1))
    h3, sh, qh = _tc_p3(edge_rep, Ra, Rb, Wa1, b_row, scale, W1t, s1, t1,
                        W2t)
    s2, t2 = _tc_stats2(sh, qh, gamma2.reshape(1, -1), beta2.reshape(1, -1))
    return _tc_p4(h3, s2, t2)


# speedup vs baseline: 1.1220x; 1.1220x over previous
import functools

import jax
import jax.numpy as jnp
from jax import lax
from jax.experimental import pallas as pl
from jax.experimental.pallas import tpu as pltpu
from jax.experimental.pallas import tpu_sc as plsc

H = 128
N = 10000
E = 320000
R2E = 2 * E
BN_EPS = 1e-5

NC, NS = 2, 16
NW = NC * NS
CH = 128
NCHUNK = R2E // CH
NK_CORE = -(-NCHUNK // NS)
NK_ALL = -(-NCHUNK // NW)
NT = 10240
NPT = NT // NS

BK = 5120
GSTEPS = R2E // BK


def _sc_scatter(edge_rep, inter, swap, znh, oneh):
    mesh = plsc.VectorSubcoreMesh(core_axis_name="c", subcore_axis_name="s")

    @functools.partial(
        pl.kernel,
        out_type=(
            jax.ShapeDtypeStruct((NT, H), jnp.float32),
            jax.ShapeDtypeStruct((NT, H), jnp.float32),
            jax.ShapeDtypeStruct((NT, H), jnp.float32),
            jax.ShapeDtypeStruct((NT, H), jnp.float32),
        ),
        mesh=mesh,
        scratch_types=[
            pltpu.VMEM_SHARED((NT, H), jnp.float32),
            pltpu.VMEM((CH, H), jnp.float32),
            pltpu.VMEM((CH, H), jnp.float32),
            pltpu.VMEM((CH,), jnp.int32),
            pltpu.VMEM((CH,), jnp.int32),
            pltpu.SemaphoreType.DMA,
            pltpu.SemaphoreType.DMA,
            pltpu.SemaphoreType.DMA,
            pltpu.SemaphoreType.DMA,
        ],
    )
    def k(er_hbm, inter_hbm, swap_hbm, znh_hbm, oneh_hbm,
          ns1_hbm, cross_hbm, dega_hbm, degb_hbm,
          tab_sh, vrows0, vrows1, vidx0, vidx1,
          sr0, sr1, si0, si1):
        c = lax.axis_index("c")
        s = lax.axis_index("s")
        wid = s * NC + c
        base = s * NPT
        vrows = vrows0

        def zero_table():
            pltpu.sync_copy(znh_hbm, vrows)
            for j in range(NPT // CH):
                pltpu.sync_copy(vrows, tab_sh.at[pl.ds(base + j * CH, CH)])

        zero_table()
        plsc.subcore_barrier()

        def make_row_phase(idx_hbm):
            bufs = ((vrows0, vidx0, sr0, si0), (vrows1, vidx1, sr1, si1))

            def start(mm, b):
                vr, vi, sr, si = bufs[b]

                @pl.when(mm < NCHUNK)
                def _():
                    r0 = mm * CH
                    pltpu.async_copy(er_hbm.at[pl.ds(r0, CH)], vr, sr)
                    pltpu.async_copy(idx_hbm.at[pl.ds(r0, CH)], vi, si)

            def finish(mm, b):
                vr, vi, sr, si = bufs[b]

                @pl.when(mm < NCHUNK)
                def _():
                    pltpu.make_async_copy(
                        er_hbm.at[pl.ds(0, CH)], vr, sr).wait()
                    pltpu.make_async_copy(
                        idx_hbm.at[pl.ds(0, CH)], vi, si).wait()
                    pltpu.sync_copy(vr, tab_sh.at[vi], add=True)

            def run():
                start(s, 0)

                def body(t2, carry):
                    m0 = s + (2 * t2) * NS
                    start(m0 + NS, 1)
                    finish(m0, 0)
                    start(m0 + 2 * NS, 0)
                    finish(m0 + NS, 1)
                    return carry

                lax.fori_loop(0, -(-NK_CORE // 2), body, 0)

            return run

        @pl.when(c == 0)
        def _():
            make_row_phase(inter_hbm)()

        @pl.when(c == 1)
        def _():
            make_row_phase(swap_hbm)()

        plsc.subcore_barrier()

        def write_table(dst_hbm):
            for j in range(NPT // CH):
                sl = pl.ds(base + j * CH, CH)
                pltpu.sync_copy(tab_sh.at[sl], vrows)
                pltpu.sync_copy(vrows, dst_hbm.at[sl])

        @pl.when(c == 0)
        def _():
            write_table(ns1_hbm)

        @pl.when(c == 1)
        def _():
            write_table(cross_hbm)

        plsc.subcore_barrier()
        zero_table()
        pltpu.sync_copy(oneh_hbm, vrows0)
        plsc.subcore_barrier()

        def make_deg_body(idx_hbm):
            def body(kk, carry):
                m = wid + kk * NW

                @pl.when(m < NCHUNK)
                def _():
                    r0 = m * CH
                    pltpu.sync_copy(idx_hbm.at[pl.ds(r0, CH)], vidx0)
                    pltpu.sync_copy(vrows0, tab_sh.at[vidx0], add=True)

                return carry

            return body

        @pl.when(c == 0)
        def _():
            lax.fori_loop(0, NK_ALL, make_deg_body(inter_hbm), 0)

        @pl.when(c == 1)
        def _():
            lax.fori_loop(0, NK_ALL, make_deg_body(swap_hbm), 0)

        plsc.subcore_barrier()

        @pl.when(c == 0)
        def _():
            write_table(dega_hbm)

        @pl.when(c == 1)
        def _():
            write_table(degb_hbm)

    return k(edge_rep, inter, swap, znh, oneh)


NRG = R2E
NCHUNK_G = NRG // CH
NKG = -(-NCHUNK_G // NW)


def _sc_gather(P, Q, inter, swap):
    mesh = plsc.VectorSubcoreMesh(core_axis_name="c", subcore_axis_name="s")

    @functools.partial(
        pl.kernel,
        out_type=jax.ShapeDtypeStruct((NRG, H), jnp.float32),
        mesh=mesh,
        scratch_types=[
            pltpu.VMEM((CH,), jnp.int32),
            pltpu.VMEM((CH,), jnp.int32),
            pltpu.VMEM((CH,), jnp.int32),
            pltpu.VMEM((CH,), jnp.int32),
            pltpu.VMEM((CH, H), jnp.float32),
            pltpu.VMEM((CH, H), jnp.float32),
            pltpu.VMEM((CH, H), jnp.float32),
            pltpu.VMEM((CH, H), jnp.float32),
            pltpu.SemaphoreType.DMA,
            pltpu.SemaphoreType.DMA,
            pltpu.SemaphoreType.DMA,
            pltpu.SemaphoreType.DMA,
        ],
    )
    def k(p_hbm, q_hbm, inter_hbm, swap_hbm, r_hbm,
          vidxa0, vidxb0, vidxa1, vidxb1, bufp0, bufq0, bufp1, bufq1,
          sema0, semb0, sema1, semb1):
        c = lax.axis_index("c")
        s = lax.axis_index("s")
        wid = s * NC + c
        bufs = ((vidxa0, vidxb0, bufp0, bufq0, sema0, semb0),
                (vidxa1, vidxb1, bufp1, bufq1, sema1, semb1))

        def start(mm, b):
            ia, ib, bp, bq, sa, sb = bufs[b]

            @pl.when(mm < NCHUNK_G)
            def _():
                r0 = mm * CH
                pltpu.sync_copy(inter_hbm.at[pl.ds(r0, CH)], ia)
                pltpu.sync_copy(swap_hbm.at[pl.ds(r0, CH)], ib)
                pltpu.async_copy(p_hbm.at[ia], bp, sa)
                pltpu.async_copy(q_hbm.at[ib], bq, sb)

        def finish(mm, b):
            ia, ib, bp, bq, sa, sb = bufs[b]

            @pl.when(mm < NCHUNK_G)
            def _():
                pltpu.make_async_copy(p_hbm.at[ia], bp, sa).wait()
                pltpu.make_async_copy(q_hbm.at[ib], bq, sb).wait()

                def addrow(r, cr):
                    for j in range(H // 16):
                        slj = pl.ds(j * 16, 16)
                        bp[r, slj] = bp[r, slj] + bq[r, slj]
                    return cr

                lax.fori_loop(0, CH, addrow, 0)
                pltpu.sync_copy(bp, r_hbm.at[pl.ds(mm * CH, CH)])

        start(wid, 0)

        def body(t2, carry):
            m0 = wid + (2 * t2) * NW
            start(m0 + NW, 1)
            finish(m0, 0)
            start(m0 + 2 * NW, 0)
            finish(m0 + NW, 1)
            return carry

        lax.fori_loop(0, -(-NKG // 2), body, 0)

    return k(P, Q, inter, swap)


def _node_body(ns1_ref, cross_ref, dega_ref, degb_ref, e0_ref,
               wc1_ref, wc12_ref, wa12_ref, p_ref, q_ref):
    ns1 = ns1_ref[...]
    dcol = jnp.dot(dega_ref[...] + degb_ref[...], e0_ref[...],
                   preferred_element_type=jnp.float32)
    p_ref[...] = (dcol *
                  jnp.dot(ns1, wc12_ref[...], preferred_element_type=jnp.float32)
                  + jnp.dot(cross_ref[...], wc1_ref[...],
                            preferred_element_type=jnp.float32))
    q_ref[...] = jnp.dot(ns1, wa12_ref[...], preferred_element_type=jnp.float32)


BNT = 2048


def _tc_node(ns1, cross, dega, degb, Wc1, Wc12, Wa12):
    e0 = jnp.zeros((H, 1), jnp.float32).at[0, 0].set(1.0)
    return pl.pallas_call(
        _node_body,
        grid=(NT // BNT,),
        in_specs=[
            pl.BlockSpec((BNT, H), lambda i: (i, 0)),
            pl.BlockSpec((BNT, H), lambda i: (i, 0)),
            pl.BlockSpec((BNT, H), lambda i: (i, 0)),
            pl.BlockSpec((BNT, H), lambda i: (i, 0)),
            pl.BlockSpec((H, 1), lambda i: (0, 0)),
            pl.BlockSpec((H, H), lambda i: (0, 0)),
            pl.BlockSpec((H, H), lambda i: (0, 0)),
            pl.BlockSpec((H, H), lambda i: (0, 0)),
        ],
        out_specs=(
            pl.BlockSpec((BNT, H), lambda i: (i, 0)),
            pl.BlockSpec((BNT, H), lambda i: (i, 0)),
        ),
        out_shape=(
            jax.ShapeDtypeStruct((NT, H), jnp.float32),
            jax.ShapeDtypeStruct((NT, H), jnp.float32),
        ),
        compiler_params=pltpu.CompilerParams(
            dimension_semantics=("parallel",)),
    )(ns1, cross, dega, degb, e0, Wc1, Wc12, Wa12)


def _p2_body(er_ref, r_ref, wa1_ref, b_ref, scale_ref,
             y_ref, sy_ref, gy_ref):
    i = pl.program_id(0)
    er = er_ref[...]
    y = (jnp.dot(er, wa1_ref[...], preferred_element_type=jnp.float32)
         + r_ref[...] + b_ref[...] + scale_ref[0, 0] * er)
    y_ref[...] = y.astype(jnp.bfloat16)

    @pl.when(i == 0)
    def _():
        sy_ref[...] = jnp.zeros_like(sy_ref)
        gy_ref[...] = jnp.zeros_like(gy_ref)

    sy_ref[...] += jnp.sum(y, axis=0, keepdims=True)
    gy_ref[...] += lax.dot_general(y, y, (((0,), (0,)), ((), ())),
                                   preferred_element_type=jnp.float32)


def _tc_p2(edge_rep, R, Wa1, b_lin, scale):
    return pl.pallas_call(
        _p2_body,
        grid=(GSTEPS,),
        in_specs=[
            pl.BlockSpec((BK, H), lambda i: (i, 0)),
            pl.BlockSpec((BK, H), lambda i: (i, 0)),
            pl.BlockSpec((H, H), lambda i: (0, 0)),
            pl.BlockSpec((1, H), lambda i: (0, 0)),
            pl.BlockSpec(memory_space=pltpu.SMEM),
        ],
        out_specs=(
            pl.BlockSpec((BK, H), lambda i: (i, 0)),
            pl.BlockSpec((1, H), lambda i: (0, 0)),
            pl.BlockSpec((H, H), lambda i: (0, 0)),
        ),
        out_shape=(
            jax.ShapeDtypeStruct((R2E, H), jnp.bfloat16),
            jax.ShapeDtypeStruct((1, H), jnp.float32),
            jax.ShapeDtypeStruct((H, H), jnp.float32),
        ),
        compiler_params=pltpu.CompilerParams(
            dimension_semantics=("arbitrary",)),
    )(edge_rep, R, Wa1, b_lin, scale)


def _stats_body(sum_ref, gram_ref, w_ref, gamma_ref, beta_ref, s_ref, t_ref):
    w = w_ref[...]
    inv_n = 1.0 / float(R2E)
    mu = jnp.dot(sum_ref[...], w, preferred_element_type=jnp.float32) * inv_n
    gw = jnp.dot(gram_ref[...], w, preferred_element_type=jnp.float32)
    e2 = jnp.sum(w * gw, axis=0, keepdims=True) * inv_n
    var = e2 - mu * mu
    s = gamma_ref[...] * lax.rsqrt(var + BN_EPS)
    s_ref[...] = s
    t_ref[...] = beta_ref[...] - mu * s


def _tc_stats(sum_v, gram, W, gamma, beta):
    d = W.shape[1]
    return pl.pallas_call(
        _stats_body,
        out_shape=(
            jax.ShapeDtypeStruct((1, d), jnp.float32),
            jax.ShapeDtypeStruct((1, d), jnp.float32),
        ),
    )(sum_v, gram, W, gamma, beta)


def _p3_body(y_ref, w1t_ref, s1_ref, t1_ref, w2t_ref,
             h3_ref, sh_ref, qh_ref):
    i = pl.program_id(0)
    h = jnp.dot(y_ref[...], w1t_ref[...], preferred_element_type=jnp.float32)
    z = jnp.maximum(h * s1_ref[...] + t1_ref[...], 0.0)
    h3 = jnp.dot(z, w2t_ref[...], preferred_element_type=jnp.float32)
    h3_ref[...] = h3.astype(jnp.bfloat16)

    @pl.when(i == 0)
    def _():
        sh_ref[...] = jnp.zeros_like(sh_ref)
        qh_ref[...] = jnp.zeros_like(qh_ref)

    sh_ref[...] += jnp.sum(h3, axis=0, keepdims=True)
    qh_ref[...] += jnp.sum(h3 * h3, axis=0, keepdims=True)


def _tc_p3(y, W1t, s1, t1, W2t):
    return pl.pallas_call(
        _p3_body,
        grid=(GSTEPS,),
        in_specs=[
            pl.BlockSpec((BK, H), lambda i: (i, 0)),
            pl.BlockSpec((H, 2 * H), lambda i: (0, 0)),
            pl.BlockSpec((1, 2 * H), lambda i: (0, 0)),
            pl.BlockSpec((1, 2 * H), lambda i: (0, 0)),
            pl.BlockSpec((2 * H, H), lambda i: (0, 0)),
        ],
        out_specs=(
            pl.BlockSpec((BK, H), lambda i: (i, 0)),
            pl.BlockSpec((1, H), lambda i: (0, 0)),
            pl.BlockSpec((1, H), lambda i: (0, 0)),
        ),
        out_shape=(
            jax.ShapeDtypeStruct((R2E, H), jnp.bfloat16),
            jax.ShapeDtypeStruct((1, H), jnp.float32),
            jax.ShapeDtypeStruct((1, H), jnp.float32),
        ),
        compiler_params=pltpu.CompilerParams(
            dimension_semantics=("arbitrary",)),
    )(y, W1t, s1, t1, W2t)


def _stats2_body(sh_ref, qh_ref, gamma_ref, beta_ref, s_ref, t_ref):
    inv_n = 1.0 / float(R2E)
    mu = sh_ref[...] * inv_n
    var = qh_ref[...] * inv_n - mu * mu
    s = gamma_ref[...] * lax.rsqrt(var + BN_EPS)
    s_ref[...] = s
    t_ref[...] = beta_ref[...] - mu * s


def _tc_stats2(sh, qh, gamma, beta):
    return pl.pallas_call(
        _stats2_body,
        out_shape=(
            jax.ShapeDtypeStruct((1, H), jnp.float32),
            jax.ShapeDtypeStruct((1, H), jnp.float32),
        ),
    )(sh, qh, gamma, beta)


def _p4_body(h3_ref, s2_ref, t2_ref, out_ref):
    h3 = h3_ref[...].astype(jnp.float32)
    out_ref[...] = jnp.maximum(h3 * s2_ref[...] + t2_ref[...], 0.0)


def _tc_p4(h3, s2, t2):
    return pl.pallas_call(
        _p4_body,
        grid=(GSTEPS,),
        in_specs=[
            pl.BlockSpec((BK, H), lambda i: (i, 0)),
            pl.BlockSpec((1, H), lambda i: (0, 0)),
            pl.BlockSpec((1, H), lambda i: (0, 0)),
        ],
        out_specs=pl.BlockSpec((BK, H), lambda i: (i, 0)),
        out_shape=jax.ShapeDtypeStruct((R2E, H), jnp.float32),
        compiler_params=pltpu.CompilerParams(
            dimension_semantics=("parallel",)),
    )(h3, s2, t2)


def kernel(edge_rep, edge_index, W_lin, b_lin, W1, gamma1, beta1,
           W2, gamma2, beta2, eps):
    src = edge_index[0]
    dst = edge_index[1]
    inter = jnp.stack([src, dst], axis=1).reshape(-1)
    swap = jnp.stack([dst, src], axis=1).reshape(-1)

    Wt = W_lin.T
    Wa1, Wa2 = Wt[0:H], Wt[H:2 * H]
    Wb1, Wb2 = Wt[2 * H:3 * H], Wt[3 * H:4 * H]
    Wc1 = Wa1 + Wb1
    Wc12 = Wc1 + Wa2 + Wb2
    Wa12 = Wa1 + Wa2

    znh = jnp.zeros((CH, H), jnp.float32)
    oneh = jnp.ones((CH, H), jnp.float32)

    ns1, cross, dega, degb = _sc_scatter(edge_rep, inter, swap, znh, oneh)
    P, Q = _tc_node(ns1, cross, dega, degb, Wc1, Wc12, Wa12)
    R = _sc_gather(P, Q, inter, swap)

    scale = (1.0 + eps).reshape(1, 1)
    y, sy, Gy = _tc_p2(edge_rep, R, Wa1, b_lin.reshape(1, H), scale)

    W1t = W1.T
    W2t = W2.T
    s1, t1 = _tc_stats(sy, Gy, W1t, gamma1.reshape(1, -1),
                       beta1.reshape(1, -1))
    h3, sh, qh = _tc_p3(y, W1t, s1, t1, W2t)
    s2, t2 = _tc_stats2(sh, qh, gamma2.reshape(1, -1), beta2.reshape(1, -1))
    return _tc_p4(h3, s2, t2)


# speedup vs baseline: 1.1312x; 1.0082x over previous
import functools

import jax
import jax.numpy as jnp
from jax import lax
from jax.experimental import pallas as pl
from jax.experimental.pallas import tpu as pltpu
from jax.experimental.pallas import tpu_sc as plsc

H = 128
N = 10000
E = 320000
R2E = 2 * E
BN_EPS = 1e-5

NC, NS = 2, 16
NW = NC * NS
CH = 128
NCHUNK = R2E // CH
NK_CORE = -(-NCHUNK // NS)
NK_ALL = -(-NCHUNK // NW)
NT = 10240
NPT = NT // NS

BK = 5120
GSTEPS = R2E // BK


def _sc_scatter(edge_rep, inter, swap, znh, oneh):
    mesh = plsc.VectorSubcoreMesh(core_axis_name="c", subcore_axis_name="s")

    @functools.partial(
        pl.kernel,
        out_type=(
            jax.ShapeDtypeStruct((NT, H), jnp.float32),
            jax.ShapeDtypeStruct((NT, H), jnp.float32),
            jax.ShapeDtypeStruct((NT, H), jnp.float32),
            jax.ShapeDtypeStruct((NT, H), jnp.float32),
        ),
        mesh=mesh,
        scratch_types=[
            pltpu.VMEM_SHARED((NT, H), jnp.float32),
            pltpu.VMEM((CH, H), jnp.float32),
            pltpu.VMEM((CH, H), jnp.float32),
            pltpu.VMEM((CH,), jnp.int32),
            pltpu.VMEM((CH,), jnp.int32),
            pltpu.SemaphoreType.DMA,
            pltpu.SemaphoreType.DMA,
            pltpu.SemaphoreType.DMA,
            pltpu.SemaphoreType.DMA,
        ],
    )
    def k(er_hbm, inter_hbm, swap_hbm, znh_hbm, oneh_hbm,
          ns1_hbm, cross_hbm, dega_hbm, degb_hbm,
          tab_sh, vrows0, vrows1, vidx0, vidx1,
          sr0, sr1, si0, si1):
        c = lax.axis_index("c")
        s = lax.axis_index("s")
        wid = s * NC + c
        base = s * NPT
        vrows = vrows0

        def zero_table():
            pltpu.sync_copy(znh_hbm, vrows)
            for j in range(NPT // CH):
                pltpu.sync_copy(vrows, tab_sh.at[pl.ds(base + j * CH, CH)])

        zero_table()
        plsc.subcore_barrier()

        def make_row_phase(idx_hbm):
            bufs = ((vrows0, vidx0, sr0, si0), (vrows1, vidx1, sr1, si1))

            def start(mm, b):
                vr, vi, sr, si = bufs[b]

                @pl.when(mm < NCHUNK)
                def _():
                    r0 = mm * CH
                    pltpu.async_copy(er_hbm.at[pl.ds(r0, CH)], vr, sr)
                    pltpu.async_copy(idx_hbm.at[pl.ds(r0, CH)], vi, si)

            def finish(mm, b):
                vr, vi, sr, si = bufs[b]

                @pl.when(mm < NCHUNK)
                def _():
                    pltpu.make_async_copy(
                        er_hbm.at[pl.ds(0, CH)], vr, sr).wait()
                    pltpu.make_async_copy(
                        idx_hbm.at[pl.ds(0, CH)], vi, si).wait()
                    pltpu.sync_copy(vr, tab_sh.at[vi], add=True)

            def run():
                start(s, 0)

                def body(t2, carry):
                    m0 = s + (2 * t2) * NS
                    start(m0 + NS, 1)
                    finish(m0, 0)
                    start(m0 + 2 * NS, 0)
                    finish(m0 + NS, 1)
                    return carry

                lax.fori_loop(0, -(-NK_CORE // 2), body, 0)

            return run

        @pl.when(c == 0)
        def _():
            make_row_phase(inter_hbm)()

        @pl.when(c == 1)
        def _():
            make_row_phase(swap_hbm)()

        plsc.subcore_barrier()

        def write_table(dst_hbm):
            for j in range(NPT // CH):
                sl = pl.ds(base + j * CH, CH)
                pltpu.sync_copy(tab_sh.at[sl], vrows)
                pltpu.sync_copy(vrows, dst_hbm.at[sl])

        @pl.when(c == 0)
        def _():
            write_table(ns1_hbm)

        @pl.when(c == 1)
        def _():
            write_table(cross_hbm)

        plsc.subcore_barrier()
        zero_table()
        pltpu.sync_copy(oneh_hbm, vrows0)
        plsc.subcore_barrier()

        def make_deg_body(idx_hbm):
            def body(kk, carry):
                m = wid + kk * NW

                @pl.when(m < NCHUNK)
                def _():
                    r0 = m * CH
                    pltpu.sync_copy(idx_hbm.at[pl.ds(r0, CH)], vidx0)
                    pltpu.sync_copy(vrows0, tab_sh.at[vidx0], add=True)

                return carry

            return body

        @pl.when(c == 0)
        def _():
            lax.fori_loop(0, NK_ALL, make_deg_body(inter_hbm), 0)

        @pl.when(c == 1)
        def _():
            lax.fori_loop(0, NK_ALL, make_deg_body(swap_hbm), 0)

        plsc.subcore_barrier()

        @pl.when(c == 0)
        def _():
            write_table(dega_hbm)

        @pl.when(c == 1)
        def _():
            write_table(degb_hbm)

    return k(edge_rep, inter, swap, znh, oneh)


NRG = R2E
NCHUNK_G = NRG // CH
NKG = -(-NCHUNK_G // NW)


def _sc_gather(P, Q, inter, swap):
    mesh = plsc.VectorSubcoreMesh(core_axis_name="c", subcore_axis_name="s")

    @functools.partial(
        pl.kernel,
        out_type=jax.ShapeDtypeStruct((NRG, H), jnp.float32),
        mesh=mesh,
        scratch_types=[
            pltpu.VMEM((CH,), jnp.int32),
            pltpu.VMEM((CH,), jnp.int32),
            pltpu.VMEM((CH,), jnp.int32),
            pltpu.VMEM((CH,), jnp.int32),
            pltpu.VMEM((CH, H), jnp.float32),
            pltpu.VMEM((CH, H), jnp.float32),
            pltpu.VMEM((CH, H), jnp.float32),
            pltpu.VMEM((CH, H), jnp.float32),
            pltpu.SemaphoreType.DMA,
            pltpu.SemaphoreType.DMA,
            pltpu.SemaphoreType.DMA,
            pltpu.SemaphoreType.DMA,
        ],
    )
    def k(p_hbm, q_hbm, inter_hbm, swap_hbm, r_hbm,
          vidxa0, vidxb0, vidxa1, vidxb1, bufp0, bufq0, bufp1, bufq1,
          sema0, semb0, sema1, semb1):
        c = lax.axis_index("c")
        s = lax.axis_index("s")
        wid = s * NC + c
        bufs = ((vidxa0, vidxb0, bufp0, bufq0, sema0, semb0),
                (vidxa1, vidxb1, bufp1, bufq1, sema1, semb1))

        def start(mm, b):
            ia, ib, bp, bq, sa, sb = bufs[b]

            @pl.when(mm < NCHUNK_G)
            def _():
                r0 = mm * CH
                pltpu.sync_copy(inter_hbm.at[pl.ds(r0, CH)], ia)
                pltpu.sync_copy(swap_hbm.at[pl.ds(r0, CH)], ib)
                pltpu.async_copy(p_hbm.at[ia], bp, sa)
                pltpu.async_copy(q_hbm.at[ib], bq, sb)

        def finish(mm, b):
            ia, ib, bp, bq, sa, sb = bufs[b]

            @pl.when(mm < NCHUNK_G)
            def _():
                pltpu.make_async_copy(p_hbm.at[ia], bp, sa).wait()
                pltpu.make_async_copy(q_hbm.at[ib], bq, sb).wait()

                def addrow(r, cr):
                    for j in range(H // 16):
                        slj = pl.ds(j * 16, 16)
                        bp[r, slj] = bp[r, slj] + bq[r, slj]
                    return cr

                lax.fori_loop(0, CH, addrow, 0)
                pltpu.sync_copy(bp, r_hbm.at[pl.ds(mm * CH, CH)])

        start(wid, 0)

        def body(t2, carry):
            m0 = wid + (2 * t2) * NW
            start(m0 + NW, 1)
            finish(m0, 0)
            start(m0 + 2 * NW, 0)
            finish(m0 + NW, 1)
            return carry

        lax.fori_loop(0, -(-NKG // 2), body, 0)

    return k(P, Q, inter, swap)


def _node_body(ns1_ref, cross_ref, dega_ref, degb_ref, e0_ref,
               wc1_ref, wc12_ref, wa12_ref, p_ref, q_ref):
    ns1 = ns1_ref[...]
    dcol = jnp.dot(dega_ref[...] + degb_ref[...], e0_ref[...],
                   preferred_element_type=jnp.float32)
    p_ref[...] = (dcol *
                  jnp.dot(ns1, wc12_ref[...], preferred_element_type=jnp.float32)
                  + jnp.dot(cross_ref[...], wc1_ref[...],
                            preferred_element_type=jnp.float32))
    q_ref[...] = jnp.dot(ns1, wa12_ref[...], preferred_element_type=jnp.float32)


BNT = 2048


def _tc_node(ns1, cross, dega, degb, Wc1, Wc12, Wa12):
    e0 = jnp.zeros((H, 1), jnp.float32).at[0, 0].set(1.0)
    return pl.pallas_call(
        _node_body,
        grid=(NT // BNT,),
        in_specs=[
            pl.BlockSpec((BNT, H), lambda i: (i, 0)),
            pl.BlockSpec((BNT, H), lambda i: (i, 0)),
            pl.BlockSpec((BNT, H), lambda i: (i, 0)),
            pl.BlockSpec((BNT, H), lambda i: (i, 0)),
            pl.BlockSpec((H, 1), lambda i: (0, 0)),
            pl.BlockSpec((H, H), lambda i: (0, 0)),
            pl.BlockSpec((H, H), lambda i: (0, 0)),
            pl.BlockSpec((H, H), lambda i: (0, 0)),
        ],
        out_specs=(
            pl.BlockSpec((BNT, H), lambda i: (i, 0)),
            pl.BlockSpec((BNT, H), lambda i: (i, 0)),
        ),
        out_shape=(
            jax.ShapeDtypeStruct((NT, H), jnp.float32),
            jax.ShapeDtypeStruct((NT, H), jnp.float32),
        ),
        compiler_params=pltpu.CompilerParams(
            dimension_semantics=("parallel",)),
    )(ns1, cross, dega, degb, e0, Wc1, Wc12, Wa12)


def _p2_body(er_ref, r_ref, wa1_ref, b_ref, scale_ref,
             y_ref, sy_ref, gy_ref):
    i = pl.program_id(0)
    er = er_ref[...]
    y = (jnp.dot(er, wa1_ref[...], preferred_element_type=jnp.float32)
         + r_ref[...] + b_ref[...] + scale_ref[0, 0] * er)
    y_ref[...] = y.astype(jnp.bfloat16)

    @pl.when(i == 0)
    def _():
        sy_ref[...] = jnp.zeros_like(sy_ref)
        gy_ref[...] = jnp.zeros_like(gy_ref)

    sy_ref[...] += jnp.sum(y, axis=0, keepdims=True)
    gy_ref[...] += lax.dot_general(y, y, (((0,), (0,)), ((), ())),
                                   preferred_element_type=jnp.float32)


def _tc_p2(edge_rep, R, Wa1, b_lin, scale):
    return pl.pallas_call(
        _p2_body,
        grid=(GSTEPS,),
        in_specs=[
            pl.BlockSpec((BK, H), lambda i: (i, 0)),
            pl.BlockSpec((BK, H), lambda i: (i, 0)),
            pl.BlockSpec((H, H), lambda i: (0, 0)),
            pl.BlockSpec((1, H), lambda i: (0, 0)),
            pl.BlockSpec(memory_space=pltpu.SMEM),
        ],
        out_specs=(
            pl.BlockSpec((BK, H), lambda i: (i, 0)),
            pl.BlockSpec((1, H), lambda i: (0, 0)),
            pl.BlockSpec((H, H), lambda i: (0, 0)),
        ),
        out_shape=(
            jax.ShapeDtypeStruct((R2E, H), jnp.bfloat16),
            jax.ShapeDtypeStruct((1, H), jnp.float32),
            jax.ShapeDtypeStruct((H, H), jnp.float32),
        ),
        compiler_params=pltpu.CompilerParams(
            dimension_semantics=("arbitrary",)),
    )(edge_rep, R, Wa1, b_lin, scale)


def _stats_body(sum_ref, gram_ref, w_ref, gamma_ref, beta_ref, s_ref, t_ref):
    w = w_ref[...]
    inv_n = 1.0 / float(R2E)
    mu = jnp.dot(sum_ref[...], w, preferred_element_type=jnp.float32) * inv_n
    gw = jnp.dot(gram_ref[...], w, preferred_element_type=jnp.float32)
    e2 = jnp.sum(w * gw, axis=0, keepdims=True) * inv_n
    var = e2 - mu * mu
    s = gamma_ref[...] * lax.rsqrt(var + BN_EPS)
    s_ref[...] = s
    t_ref[...] = beta_ref[...] - mu * s


def _tc_stats(sum_v, gram, W, gamma, beta):
    d = W.shape[1]
    return pl.pallas_call(
        _stats_body,
        out_shape=(
            jax.ShapeDtypeStruct((1, d), jnp.float32),
            jax.ShapeDtypeStruct((1, d), jnp.float32),
        ),
    )(sum_v, gram, W, gamma, beta)


def _p3_body(y_ref, sy_ref, gy_ref, g1_ref, b1_ref, w1t_ref, w2t_ref,
             h3_ref, sh_ref, qh_ref, st_ref):
    i = pl.program_id(0)
    w1t = w1t_ref[...]

    @pl.when(i == 0)
    def _():
        inv_n = 1.0 / float(R2E)
        mu = jnp.dot(sy_ref[...], w1t,
                     preferred_element_type=jnp.float32) * inv_n
        gw = jnp.dot(gy_ref[...], w1t, preferred_element_type=jnp.float32)
        e2 = jnp.sum(w1t * gw, axis=0, keepdims=True) * inv_n
        var = e2 - mu * mu
        s1 = g1_ref[...] * lax.rsqrt(var + BN_EPS)
        st_ref[0:1] = s1
        st_ref[1:2] = b1_ref[...] - mu * s1
        sh_ref[...] = jnp.zeros_like(sh_ref)
        qh_ref[...] = jnp.zeros_like(qh_ref)

    h = jnp.dot(y_ref[...], w1t, preferred_element_type=jnp.float32)
    z = jnp.maximum(h * st_ref[0:1] + st_ref[1:2], 0.0)
    h3 = jnp.dot(z, w2t_ref[...], preferred_element_type=jnp.float32)
    h3_ref[...] = h3.astype(jnp.bfloat16)
    sh_ref[...] += jnp.sum(h3, axis=0, keepdims=True)
    qh_ref[...] += jnp.sum(h3 * h3, axis=0, keepdims=True)


def _tc_p3(y, sy, Gy, gamma1, beta1, W1t, W2t):
    return pl.pallas_call(
        _p3_body,
        grid=(GSTEPS,),
        in_specs=[
            pl.BlockSpec((BK, H), lambda i: (i, 0)),
            pl.BlockSpec((1, H), lambda i: (0, 0)),
            pl.BlockSpec((H, H), lambda i: (0, 0)),
            pl.BlockSpec((1, 2 * H), lambda i: (0, 0)),
            pl.BlockSpec((1, 2 * H), lambda i: (0, 0)),
            pl.BlockSpec((H, 2 * H), lambda i: (0, 0)),
            pl.BlockSpec((2 * H, H), lambda i: (0, 0)),
        ],
        out_specs=(
            pl.BlockSpec((BK, H), lambda i: (i, 0)),
            pl.BlockSpec((1, H), lambda i: (0, 0)),
            pl.BlockSpec((1, H), lambda i: (0, 0)),
        ),
        out_shape=(
            jax.ShapeDtypeStruct((R2E, H), jnp.bfloat16),
            jax.ShapeDtypeStruct((1, H), jnp.float32),
            jax.ShapeDtypeStruct((1, H), jnp.float32),
        ),
        scratch_shapes=[pltpu.VMEM((2, 2 * H), jnp.float32)],
        compiler_params=pltpu.CompilerParams(
            dimension_semantics=("arbitrary",)),
    )(y, sy, Gy, gamma1, beta1, W1t, W2t)


def _p4_body(h3_ref, sh_ref, qh_ref, g2_ref, b2_ref, out_ref, st_ref):
    i = pl.program_id(0)

    @pl.when(i == 0)
    def _():
        inv_n = 1.0 / float(R2E)
        mu = sh_ref[...] * inv_n
        var = qh_ref[...] * inv_n - mu * mu
        s2 = g2_ref[...] * lax.rsqrt(var + BN_EPS)
        st_ref[0:1] = s2
        st_ref[1:2] = b2_ref[...] - mu * s2

    h3 = h3_ref[...].astype(jnp.float32)
    out_ref[...] = jnp.maximum(h3 * st_ref[0:1] + st_ref[1:2], 0.0)


def _tc_p4(h3, sh, qh, gamma2, beta2):
    return pl.pallas_call(
        _p4_body,
        grid=(GSTEPS,),
        in_specs=[
            pl.BlockSpec((BK, H), lambda i: (i, 0)),
            pl.BlockSpec((1, H), lambda i: (0, 0)),
            pl.BlockSpec((1, H), lambda i: (0, 0)),
            pl.BlockSpec((1, H), lambda i: (0, 0)),
            pl.BlockSpec((1, H), lambda i: (0, 0)),
        ],
        out_specs=pl.BlockSpec((BK, H), lambda i: (i, 0)),
        out_shape=jax.ShapeDtypeStruct((R2E, H), jnp.float32),
        scratch_shapes=[pltpu.VMEM((2, H), jnp.float32)],
        compiler_params=pltpu.CompilerParams(
            dimension_semantics=("arbitrary",)),
    )(h3, sh, qh, gamma2, beta2)


def kernel(edge_rep, edge_index, W_lin, b_lin, W1, gamma1, beta1,
           W2, gamma2, beta2, eps):
    src = edge_index[0]
    dst = edge_index[1]
    inter = jnp.stack([src, dst], axis=1).reshape(-1)
    swap = jnp.stack([dst, src], axis=1).reshape(-1)

    Wt = W_lin.T
    Wa1, Wa2 = Wt[0:H], Wt[H:2 * H]
    Wb1, Wb2 = Wt[2 * H:3 * H], Wt[3 * H:4 * H]
    Wc1 = Wa1 + Wb1
    Wc12 = Wc1 + Wa2 + Wb2
    Wa12 = Wa1 + Wa2

    znh = jnp.zeros((CH, H), jnp.float32)
    oneh = jnp.ones((CH, H), jnp.float32)

    ns1, cross, dega, degb = _sc_scatter(edge_rep, inter, swap, znh, oneh)
    P, Q = _tc_node(ns1, cross, dega, degb, Wc1, Wc12, Wa12)
    R = _sc_gather(P, Q, inter, swap)

    scale = (1.0 + eps).reshape(1, 1)
    y, sy, Gy = _tc_p2(edge_rep, R, Wa1, b_lin.reshape(1, H), scale)

    W1t = W1.T
    W2t = W2.T
    h3, sh, qh = _tc_p3(y, sy, Gy, gamma1.reshape(1, -1),
                        beta1.reshape(1, -1), W1t, W2t)
    return _tc_p4(h3, sh, qh, gamma2.reshape(1, -1), beta2.reshape(1, -1))


# speedup vs baseline: 1.1513x; 1.0178x over previous
import functools

import jax
import jax.numpy as jnp
from jax import lax
from jax.experimental import pallas as pl
from jax.experimental.pallas import tpu as pltpu
from jax.experimental.pallas import tpu_sc as plsc

H = 128
N = 10000
E = 320000
R2E = 2 * E
BN_EPS = 1e-5

NC, NS = 2, 16
NW = NC * NS
CH = 128
NCHUNK = R2E // CH
NK_CORE = -(-NCHUNK // NS)
NK_ALL = -(-NCHUNK // NW)
NT = 10240
NPT = NT // NS

BK = 6400
GSTEPS = R2E // BK


def _sc_scatter(edge_rep, inter, swap, znh, oneh):
    mesh = plsc.VectorSubcoreMesh(core_axis_name="c", subcore_axis_name="s")

    @functools.partial(
        pl.kernel,
        out_type=(
            jax.ShapeDtypeStruct((NT, H), jnp.float32),
            jax.ShapeDtypeStruct((NT, H), jnp.float32),
            jax.ShapeDtypeStruct((NT, H), jnp.float32),
            jax.ShapeDtypeStruct((NT, H), jnp.float32),
        ),
        mesh=mesh,
        scratch_types=[
            pltpu.VMEM_SHARED((NT, H), jnp.float32),
            pltpu.VMEM((CH, H), jnp.float32),
            pltpu.VMEM((CH, H), jnp.float32),
            pltpu.VMEM((CH,), jnp.int32),
            pltpu.VMEM((CH,), jnp.int32),
            pltpu.SemaphoreType.DMA,
            pltpu.SemaphoreType.DMA,
            pltpu.SemaphoreType.DMA,
            pltpu.SemaphoreType.DMA,
        ],
    )
    def k(er_hbm, inter_hbm, swap_hbm, znh_hbm, oneh_hbm,
          ns1_hbm, cross_hbm, dega_hbm, degb_hbm,
          tab_sh, vrows0, vrows1, vidx0, vidx1,
          sr0, sr1, si0, si1):
        c = lax.axis_index("c")
        s = lax.axis_index("s")
        wid = s * NC + c
        base = s * NPT
        vrows = vrows0

        def zero_table():
            pltpu.sync_copy(znh_hbm, vrows)
            for j in range(NPT // CH):
                pltpu.sync_copy(vrows, tab_sh.at[pl.ds(base + j * CH, CH)])

        zero_table()
        plsc.subcore_barrier()

        def make_row_phase(idx_hbm):
            bufs = ((vrows0, vidx0, sr0, si0), (vrows1, vidx1, sr1, si1))

            def start(mm, b):
                vr, vi, sr, si = bufs[b]

                @pl.when(mm < NCHUNK)
                def _():
                    r0 = mm * CH
                    pltpu.async_copy(er_hbm.at[pl.ds(r0, CH)], vr, sr)
                    pltpu.async_copy(idx_hbm.at[pl.ds(r0, CH)], vi, si)

            def finish(mm, b):
                vr, vi, sr, si = bufs[b]

                @pl.when(mm < NCHUNK)
                def _():
                    pltpu.make_async_copy(
                        er_hbm.at[pl.ds(0, CH)], vr, sr).wait()
                    pltpu.make_async_copy(
                        idx_hbm.at[pl.ds(0, CH)], vi, si).wait()
                    pltpu.sync_copy(vr, tab_sh.at[vi], add=True)

            def run():
                start(s, 0)

                def body(t2, carry):
                    m0 = s + (2 * t2) * NS
                    start(m0 + NS, 1)
                    finish(m0, 0)
                    start(m0 + 2 * NS, 0)
                    finish(m0 + NS, 1)
                    return carry

                lax.fori_loop(0, -(-NK_CORE // 2), body, 0)

            return run

        @pl.when(c == 0)
        def _():
            make_row_phase(inter_hbm)()

        @pl.when(c == 1)
        def _():
            make_row_phase(swap_hbm)()

        plsc.subcore_barrier()

        def write_table(dst_hbm):
            for j in range(NPT // CH):
                sl = pl.ds(base + j * CH, CH)
                pltpu.sync_copy(tab_sh.at[sl], vrows)
                pltpu.sync_copy(vrows, dst_hbm.at[sl])

        @pl.when(c == 0)
        def _():
            write_table(ns1_hbm)

        @pl.when(c == 1)
        def _():
            write_table(cross_hbm)

        plsc.subcore_barrier()
        zero_table()
        pltpu.sync_copy(oneh_hbm, vrows0)
        plsc.subcore_barrier()

        def make_deg_body(idx_hbm):
            def body(kk, carry):
                m = wid + kk * NW

                @pl.when(m < NCHUNK)
                def _():
                    r0 = m * CH
                    pltpu.sync_copy(idx_hbm.at[pl.ds(r0, CH)], vidx0)
                    pltpu.sync_copy(vrows0, tab_sh.at[vidx0], add=True)

                return carry

            return body

        @pl.when(c == 0)
        def _():
            lax.fori_loop(0, NK_ALL, make_deg_body(inter_hbm), 0)

        @pl.when(c == 1)
        def _():
            lax.fori_loop(0, NK_ALL, make_deg_body(swap_hbm), 0)

        plsc.subcore_barrier()

        @pl.when(c == 0)
        def _():
            write_table(dega_hbm)

        @pl.when(c == 1)
        def _():
            write_table(degb_hbm)

    return k(edge_rep, inter, swap, znh, oneh)


NRG = R2E
NCHUNK_G = NRG // CH
NKG = -(-NCHUNK_G // NW)


def _sc_gather(P, Q, inter, swap):
    mesh = plsc.VectorSubcoreMesh(core_axis_name="c", subcore_axis_name="s")

    @functools.partial(
        pl.kernel,
        out_type=jax.ShapeDtypeStruct((NRG, H), jnp.float32),
        mesh=mesh,
        scratch_types=[
            pltpu.VMEM((CH,), jnp.int32),
            pltpu.VMEM((CH,), jnp.int32),
            pltpu.VMEM((CH,), jnp.int32),
            pltpu.VMEM((CH,), jnp.int32),
            pltpu.VMEM((CH, H), jnp.float32),
            pltpu.VMEM((CH, H), jnp.float32),
            pltpu.VMEM((CH, H), jnp.float32),
            pltpu.VMEM((CH, H), jnp.float32),
            pltpu.SemaphoreType.DMA,
            pltpu.SemaphoreType.DMA,
            pltpu.SemaphoreType.DMA,
            pltpu.SemaphoreType.DMA,
        ],
    )
    def k(p_hbm, q_hbm, inter_hbm, swap_hbm, r_hbm,
          vidxa0, vidxb0, vidxa1, vidxb1, bufp0, bufq0, bufp1, bufq1,
          sema0, semb0, sema1, semb1):
        c = lax.axis_index("c")
        s = lax.axis_index("s")
        wid = s * NC + c
        bufs = ((vidxa0, vidxb0, bufp0, bufq0, sema0, semb0),
                (vidxa1, vidxb1, bufp1, bufq1, sema1, semb1))

        def start(mm, b):
            ia, ib, bp, bq, sa, sb = bufs[b]

            @pl.when(mm < NCHUNK_G)
            def _():
                r0 = mm * CH
                pltpu.sync_copy(inter_hbm.at[pl.ds(r0, CH)], ia)
                pltpu.sync_copy(swap_hbm.at[pl.ds(r0, CH)], ib)
                pltpu.async_copy(p_hbm.at[ia], bp, sa)
                pltpu.async_copy(q_hbm.at[ib], bq, sb)

        def finish(mm, b):
            ia, ib, bp, bq, sa, sb = bufs[b]

            @pl.when(mm < NCHUNK_G)
            def _():
                pltpu.make_async_copy(p_hbm.at[ia], bp, sa).wait()
                pltpu.make_async_copy(q_hbm.at[ib], bq, sb).wait()

                def addrow(r, cr):
                    for j in range(H // 16):
                        slj = pl.ds(j * 16, 16)
                        bp[r, slj] = bp[r, slj] + bq[r, slj]
                    return cr

                lax.fori_loop(0, CH, addrow, 0)
                pltpu.sync_copy(bp, r_hbm.at[pl.ds(mm * CH, CH)])

        start(wid, 0)

        def body(t2, carry):
            m0 = wid + (2 * t2) * NW
            start(m0 + NW, 1)
            finish(m0, 0)
            start(m0 + 2 * NW, 0)
            finish(m0 + NW, 1)
            return carry

        lax.fori_loop(0, -(-NKG // 2), body, 0)

    return k(P, Q, inter, swap)


def _node_body(ns1_ref, cross_ref, dega_ref, degb_ref, e0_ref,
               wc1_ref, wc12_ref, wa12_ref, p_ref, q_ref):
    ns1 = ns1_ref[...]
    dcol = jnp.dot(dega_ref[...] + degb_ref[...], e0_ref[...],
                   preferred_element_type=jnp.float32)
    p_ref[...] = (dcol *
                  jnp.dot(ns1, wc12_ref[...], preferred_element_type=jnp.float32)
                  + jnp.dot(cross_ref[...], wc1_ref[...],
                            preferred_element_type=jnp.float32))
    q_ref[...] = jnp.dot(ns1, wa12_ref[...], preferred_element_type=jnp.float32)


BNT = 2048


def _tc_node(ns1, cross, dega, degb, Wc1, Wc12, Wa12):
    e0 = jnp.zeros((H, 1), jnp.float32).at[0, 0].set(1.0)
    return pl.pallas_call(
        _node_body,
        grid=(NT // BNT,),
        in_specs=[
            pl.BlockSpec((BNT, H), lambda i: (i, 0)),
            pl.BlockSpec((BNT, H), lambda i: (i, 0)),
            pl.BlockSpec((BNT, H), lambda i: (i, 0)),
            pl.BlockSpec((BNT, H), lambda i: (i, 0)),
            pl.BlockSpec((H, 1), lambda i: (0, 0)),
            pl.BlockSpec((H, H), lambda i: (0, 0)),
            pl.BlockSpec((H, H), lambda i: (0, 0)),
            pl.BlockSpec((H, H), lambda i: (0, 0)),
        ],
        out_specs=(
            pl.BlockSpec((BNT, H), lambda i: (i, 0)),
            pl.BlockSpec((BNT, H), lambda i: (i, 0)),
        ),
        out_shape=(
            jax.ShapeDtypeStruct((NT, H), jnp.float32),
            jax.ShapeDtypeStruct((NT, H), jnp.float32),
        ),
        compiler_params=pltpu.CompilerParams(
            dimension_semantics=("parallel",)),
    )(ns1, cross, dega, degb, e0, Wc1, Wc12, Wa12)


def _p2_body(er_ref, r_ref, wa1_ref, b_ref, scale_ref,
             y_ref, sy_ref, gy_ref):
    i = pl.program_id(0)
    er = er_ref[...]
    y = (jnp.dot(er, wa1_ref[...], preferred_element_type=jnp.float32)
         + r_ref[...] + b_ref[...] + scale_ref[0, 0] * er)
    y_ref[...] = y.astype(jnp.bfloat16)

    @pl.when(i == 0)
    def _():
        sy_ref[...] = jnp.zeros_like(sy_ref)
        gy_ref[...] = jnp.zeros_like(gy_ref)

    sy_ref[...] += jnp.sum(y, axis=0, keepdims=True)
    gy_ref[...] += lax.dot_general(y, y, (((0,), (0,)), ((), ())),
                                   preferred_element_type=jnp.float32)


def _tc_p2(edge_rep, R, Wa1, b_lin, scale):
    return pl.pallas_call(
        _p2_body,
        grid=(GSTEPS,),
        in_specs=[
            pl.BlockSpec((BK, H), lambda i: (i, 0)),
            pl.BlockSpec((BK, H), lambda i: (i, 0)),
            pl.BlockSpec((H, H), lambda i: (0, 0)),
            pl.BlockSpec((1, H), lambda i: (0, 0)),
            pl.BlockSpec(memory_space=pltpu.SMEM),
        ],
        out_specs=(
            pl.BlockSpec((BK, H), lambda i: (i, 0)),
            pl.BlockSpec((1, H), lambda i: (0, 0)),
            pl.BlockSpec((H, H), lambda i: (0, 0)),
        ),
        out_shape=(
            jax.ShapeDtypeStruct((R2E, H), jnp.bfloat16),
            jax.ShapeDtypeStruct((1, H), jnp.float32),
            jax.ShapeDtypeStruct((H, H), jnp.float32),
        ),
        compiler_params=pltpu.CompilerParams(
            dimension_semantics=("arbitrary",)),
    )(edge_rep, R, Wa1, b_lin, scale)


def _stats_body(sum_ref, gram_ref, w_ref, gamma_ref, beta_ref, s_ref, t_ref):
    w = w_ref[...]
    inv_n = 1.0 / float(R2E)
    mu = jnp.dot(sum_ref[...], w, preferred_element_type=jnp.float32) * inv_n
    gw = jnp.dot(gram_ref[...], w, preferred_element_type=jnp.float32)
    e2 = jnp.sum(w * gw, axis=0, keepdims=True) * inv_n
    var = e2 - mu * mu
    s = gamma_ref[...] * lax.rsqrt(var + BN_EPS)
    s_ref[...] = s
    t_ref[...] = beta_ref[...] - mu * s


def _tc_stats(sum_v, gram, W, gamma, beta):
    d = W.shape[1]
    return pl.pallas_call(
        _stats_body,
        out_shape=(
            jax.ShapeDtypeStruct((1, d), jnp.float32),
            jax.ShapeDtypeStruct((1, d), jnp.float32),
        ),
    )(sum_v, gram, W, gamma, beta)


def _p3_body(y_ref, sy_ref, gy_ref, g1_ref, b1_ref, w1t_ref, w2t_ref,
             h3_ref, sh_ref, qh_ref, st_ref):
    i = pl.program_id(0)
    w1t = w1t_ref[...]

    @pl.when(i == 0)
    def _():
        inv_n = 1.0 / float(R2E)
        mu = jnp.dot(sy_ref[...], w1t,
                     preferred_element_type=jnp.float32) * inv_n
        gw = jnp.dot(gy_ref[...], w1t, preferred_element_type=jnp.float32)
        e2 = jnp.sum(w1t * gw, axis=0, keepdims=True) * inv_n
        var = e2 - mu * mu
        s1 = g1_ref[...] * lax.rsqrt(var + BN_EPS)
        st_ref[0:1] = s1
        st_ref[1:2] = b1_ref[...] - mu * s1
        sh_ref[...] = jnp.zeros_like(sh_ref)
        qh_ref[...] = jnp.zeros_like(qh_ref)

    h = jnp.dot(y_ref[...], w1t, preferred_element_type=jnp.float32)
    z = jnp.maximum(h * st_ref[0:1] + st_ref[1:2], 0.0)
    h3 = jnp.dot(z, w2t_ref[...], preferred_element_type=jnp.float32)
    h3_ref[...] = h3.astype(jnp.bfloat16)
    sh_ref[...] += jnp.sum(h3, axis=0, keepdims=True)
    qh_ref[...] += jnp.sum(h3 * h3, axis=0, keepdims=True)


def _tc_p3(y, sy, Gy, gamma1, beta1, W1t, W2t):
    return pl.pallas_call(
        _p3_body,
        grid=(GSTEPS,),
        in_specs=[
            pl.BlockSpec((BK, H), lambda i: (i, 0)),
            pl.BlockSpec((1, H), lambda i: (0, 0)),
            pl.BlockSpec((H, H), lambda i: (0, 0)),
            pl.BlockSpec((1, 2 * H), lambda i: (0, 0)),
            pl.BlockSpec((1, 2 * H), lambda i: (0, 0)),
            pl.BlockSpec((H, 2 * H), lambda i: (0, 0)),
            pl.BlockSpec((2 * H, H), lambda i: (0, 0)),
        ],
        out_specs=(
            pl.BlockSpec((BK, H), lambda i: (i, 0)),
            pl.BlockSpec((1, H), lambda i: (0, 0)),
            pl.BlockSpec((1, H), lambda i: (0, 0)),
        ),
        out_shape=(
            jax.ShapeDtypeStruct((R2E, H), jnp.bfloat16),
            jax.ShapeDtypeStruct((1, H), jnp.float32),
            jax.ShapeDtypeStruct((1, H), jnp.float32),
        ),
        scratch_shapes=[pltpu.VMEM((2, 2 * H), jnp.float32)],
        compiler_params=pltpu.CompilerParams(
            dimension_semantics=("arbitrary",)),
    )(y, sy, Gy, gamma1, beta1, W1t, W2t)


def _p4_body(h3_ref, sh_ref, qh_ref, g2_ref, b2_ref, out_ref, st_ref):
    i = pl.program_id(0)

    @pl.when(i == 0)
    def _():
        inv_n = 1.0 / float(R2E)
        mu = sh_ref[...] * inv_n
        var = qh_ref[...] * inv_n - mu * mu
        s2 = g2_ref[...] * lax.rsqrt(var + BN_EPS)
        st_ref[0:1] = s2
        st_ref[1:2] = b2_ref[...] - mu * s2

    h3 = h3_ref[...].astype(jnp.float32)
    out_ref[...] = jnp.maximum(h3 * st_ref[0:1] + st_ref[1:2], 0.0)


def _tc_p4(h3, sh, qh, gamma2, beta2):
    return pl.pallas_call(
        _p4_body,
        grid=(GSTEPS,),
        in_specs=[
            pl.BlockSpec((BK, H), lambda i: (i, 0)),
            pl.BlockSpec((1, H), lambda i: (0, 0)),
            pl.BlockSpec((1, H), lambda i: (0, 0)),
            pl.BlockSpec((1, H), lambda i: (0, 0)),
            pl.BlockSpec((1, H), lambda i: (0, 0)),
        ],
        out_specs=pl.BlockSpec((BK, H), lambda i: (i, 0)),
        out_shape=jax.ShapeDtypeStruct((R2E, H), jnp.float32),
        scratch_shapes=[pltpu.VMEM((2, H), jnp.float32)],
        compiler_params=pltpu.CompilerParams(
            dimension_semantics=("arbitrary",)),
    )(h3, sh, qh, gamma2, beta2)


def kernel(edge_rep, edge_index, W_lin, b_lin, W1, gamma1, beta1,
           W2, gamma2, beta2, eps):
    src = edge_index[0]
    dst = edge_index[1]
    inter = jnp.stack([src, dst], axis=1).reshape(-1)
    swap = jnp.stack([dst, src], axis=1).reshape(-1)

    Wt = W_lin.T
    Wa1, Wa2 = Wt[0:H], Wt[H:2 * H]
    Wb1, Wb2 = Wt[2 * H:3 * H], Wt[3 * H:4 * H]
    Wc1 = Wa1 + Wb1
    Wc12 = Wc1 + Wa2 + Wb2
    Wa12 = Wa1 + Wa2

    znh = jnp.zeros((CH, H), jnp.float32)
    oneh = jnp.ones((CH, H), jnp.float32)

    ns1, cross, dega, degb = _sc_scatter(edge_rep, inter, swap, znh, oneh)
    P, Q = _tc_node(ns1, cross, dega, degb, Wc1, Wc12, Wa12)
    R = _sc_gather(P, Q, inter, swap)

    scale = (1.0 + eps).reshape(1, 1)
    y, sy, Gy = _tc_p2(edge_rep, R, Wa1, b_lin.reshape(1, H), scale)

    W1t = W1.T
    W2t = W2.T
    h3, sh, qh = _tc_p3(y, sy, Gy, gamma1.reshape(1, -1),
                        beta1.reshape(1, -1), W1t, W2t)
    return _tc_p4(h3, sh, qh, gamma2.reshape(1, -1), beta2.reshape(1, -1))
